# baseline (speedup 1.0000x reference)
import functools

import jax
import jax.numpy as jnp
from jax import lax
from jax.experimental import pallas as pl
from jax.experimental.pallas import tpu as pltpu
from jax.experimental.pallas import tpu_sc as plsc

DIM = 4096
BATCH = 16384
NUM_STEPS = 4

_NC, _NS = 2, 16
_NW = _NC * _NS
_BPW = BATCH // _NW
_K = 16
_NCHUNK = _BPW // _K


@functools.partial(
    pl.kernel,
    out_type=jax.ShapeDtypeStruct((BATCH, DIM), jnp.float32),
    mesh=plsc.VectorSubcoreMesh(core_axis_name="c", subcore_axis_name="s"),
    scratch_types=[
        pltpu.VMEM((_BPW,), jnp.int32),
        pltpu.VMEM((_K, DIM), jnp.float32),
        pltpu.SemaphoreType.DMA,
    ],
)
def _sc_gather(table_hbm, ids_hbm, out_hbm, idx_v, rows_v, gsem):
    wid = lax.axis_index("s") * _NC + lax.axis_index("c")
    base = wid * _BPW
    pltpu.sync_copy(ids_hbm.at[pl.ds(base, _BPW)], idx_v)

    def body(c, carry):
        off = c * _K
        pltpu.async_copy(
            table_hbm.at[idx_v.at[pl.ds(off, _K)]], rows_v, gsem
        ).wait()
        pltpu.sync_copy(rows_v, out_hbm.at[pl.ds(base + off, _K)])
        return carry

    lax.fori_loop(0, _NCHUNK, body, 0)


def kernel(step_ids, step_embeddings):
    ids = step_ids.astype(jnp.int32)
    out = _sc_gather(step_embeddings, ids)
    return out[:, None, :]
